# Initial kernel scaffold; baseline (speedup 1.0000x reference)
import functools

import jax
import jax.numpy as jnp
from jax import lax
from jax.experimental import pallas as pl
from jax.experimental.pallas import tpu as pltpu
from jax.experimental.pallas import tpu_sc as plsc

N_NODES = 50000
N_EDGES = 800000
DIM = 64
HALF = 32
NC = 2
NS = 16
CHUNK = 128
N_CHUNKS = N_EDGES // CHUNK
STRIPE = N_NODES // NS
ZROWS = 125


def _layer_body(row_hbm, col_hbm, w_hbm, h_hbm, out_hbm,
                idx_v, dst_v, w_v, rows_v, zero_v, acc_sh, sem):
    c = lax.axis_index("c")
    s = lax.axis_index("s")

    zeros16 = jnp.zeros((16,), jnp.float32)

    def zfill(j, _):
        zero_v[j, pl.ds(0, 16)] = zeros16
        zero_v[j, pl.ds(16, 16)] = zeros16
        return 0
    lax.fori_loop(0, ZROWS, zfill, 0)

    def zacc(i, _):
        pltpu.sync_copy(zero_v.at[pl.ds(0, ZROWS)],
                        acc_sh.at[pl.ds(s * STRIPE + i * ZROWS, ZROWS)])
        return 0
    lax.fori_loop(0, STRIPE // ZROWS, zacc, 0)
    plsc.subcore_barrier()

    base_off = c * N_NODES

    def chunk_body(i, _):
        e0 = (s + i * NS) * CHUNK
        pltpu.sync_copy(col_hbm.at[pl.ds(e0, CHUNK)], idx_v)
        pltpu.sync_copy(row_hbm.at[pl.ds(e0, CHUNK)], dst_v)
        pltpu.sync_copy(w_hbm.at[pl.ds(e0, CHUNK)], w_v)

        def offs(g, _):
            idx_v[pl.ds(g * 16, 16)] = idx_v[pl.ds(g * 16, 16)] + base_off
            return 0
        lax.fori_loop(0, CHUNK // 16, offs, 0)

        pltpu.async_copy(h_hbm.at[idx_v], rows_v, sem).wait()

        def scale(j, _):
            w = w_v[j]
            rows_v[j, pl.ds(0, 16)] = rows_v[j, pl.ds(0, 16)] * w
            rows_v[j, pl.ds(16, 16)] = rows_v[j, pl.ds(16, 16)] * w
            return 0
        lax.fori_loop(0, CHUNK, scale, 0)

        pltpu.sync_copy(rows_v, acc_sh.at[dst_v], add=True)
        return 0

    n_mine = N_CHUNKS // NS + (s < (N_CHUNKS % NS)).astype(jnp.int32)
    lax.fori_loop(0, n_mine, chunk_body, 0)
    plsc.subcore_barrier()

    pltpu.sync_copy(acc_sh.at[pl.ds(s * STRIPE, STRIPE)],
                    out_hbm.at[pl.ds(base_off + s * STRIPE, STRIPE)])


@jax.jit
def _layer(row, col, w, h):
    return pl.kernel(
        _layer_body,
        out_type=jax.ShapeDtypeStruct((NC * N_NODES, HALF), jnp.float32),
        mesh=plsc.VectorSubcoreMesh(core_axis_name="c", subcore_axis_name="s"),
        scratch_types=[
            pltpu.VMEM((CHUNK,), jnp.int32),
            pltpu.VMEM((CHUNK,), jnp.int32),
            pltpu.VMEM((CHUNK,), jnp.float32),
            pltpu.VMEM((CHUNK, HALF), jnp.float32),
            pltpu.VMEM((ZROWS, HALF), jnp.float32),
            pltpu.VMEM_SHARED((N_NODES, HALF), jnp.float32),
            pltpu.SemaphoreType.DMA,
        ],
    )(row, col, w, h)


def _combine_body(h0, h1, h2, o):
    s = (h0[...] + h1[...] + h2[...]) * (1.0 / 3.0)
    o[:, :HALF] = s[0]
    o[:, HALF:] = s[1]


@jax.jit
def _combine(h0, h1, h2):
    br = 1000
    spec = pl.BlockSpec((2, br, HALF), lambda i: (0, i, 0))
    return pl.pallas_call(
        _combine_body,
        out_shape=jax.ShapeDtypeStruct((N_NODES, DIM), jnp.float32),
        grid=(N_NODES // br,),
        in_specs=[spec, spec, spec],
        out_specs=pl.BlockSpec((br, DIM), lambda i: (i, 0)),
    )(h0.reshape(NC, N_NODES, HALF), h1.reshape(NC, N_NODES, HALF),
      h2.reshape(NC, N_NODES, HALF))


def kernel(edge_index, edge_weight, user_emb, item_emb):
    row = edge_index[0].astype(jnp.int32)
    col = edge_index[1].astype(jnp.int32)
    emb = jnp.concatenate([user_emb, item_emb], axis=0)
    h0 = emb.reshape(N_NODES, NC, HALF).transpose(1, 0, 2).reshape(NC * N_NODES, HALF)
    h1 = _layer(row, col, edge_weight, h0)
    h2 = _layer(row, col, edge_weight, h1)
    return _combine(h0, h1, h2)


# speedup vs baseline: 3.4338x; 3.4338x over previous
import functools

import jax
import jax.numpy as jnp
from jax import lax
from jax.experimental import pallas as pl
from jax.experimental.pallas import tpu as pltpu
from jax.experimental.pallas import tpu_sc as plsc

N_NODES = 50000
N_PAD = 50048
N_EDGES = 800000
DIM = 64
HALF = 32
NC = 2
NS = 16
CHUNK = 128
N_CHUNKS = N_EDGES // CHUNK
STRIPE = N_PAD // NS
ZROWS = 136


def _layer_body(row_hbm, col_hbm, w_hbm, h_hbm, out_hbm,
                idx_v, dst_v, w_v, rows_v, zero_v, acc_sh, sem):
    c = lax.axis_index("c")
    s = lax.axis_index("s")

    zeros16 = jnp.zeros((16,), jnp.float32)

    def zfill(j, _):
        zero_v[j, pl.ds(0, 16)] = zeros16
        zero_v[j, pl.ds(16, 16)] = zeros16
        return 0
    lax.fori_loop(0, ZROWS, zfill, 0)

    def zacc(i, _):
        pltpu.sync_copy(zero_v.at[pl.ds(0, ZROWS)],
                        acc_sh.at[pl.ds(s * STRIPE + i * ZROWS, ZROWS)])
        return 0
    lax.fori_loop(0, STRIPE // ZROWS, zacc, 0)
    plsc.subcore_barrier()

    base_off = c * N_PAD

    def chunk_body(i, _):
        e0 = (s + i * NS) * CHUNK
        pltpu.sync_copy(col_hbm.at[pl.ds(e0, CHUNK)], idx_v)
        pltpu.sync_copy(row_hbm.at[pl.ds(e0, CHUNK)], dst_v)
        pltpu.sync_copy(w_hbm.at[pl.ds(e0, CHUNK)], w_v)

        def offs(g, _):
            idx_v[pl.ds(g * 16, 16)] = idx_v[pl.ds(g * 16, 16)] + base_off
            return 0
        lax.fori_loop(0, CHUNK // 16, offs, 0)

        pltpu.async_copy(h_hbm.at[idx_v], rows_v, sem).wait()

        def scale_group(g, _):
            base = g * 16
            wv = w_v[pl.ds(base, 16)]
            for j in range(16):
                wj = wv[j]
                r = base + j
                rows_v[r, pl.ds(0, 16)] = rows_v[r, pl.ds(0, 16)] * wj
                rows_v[r, pl.ds(16, 16)] = rows_v[r, pl.ds(16, 16)] * wj
            return 0
        lax.fori_loop(0, CHUNK // 16, scale_group, 0)

        pltpu.sync_copy(rows_v, acc_sh.at[dst_v], add=True)
        return 0

    n_mine = N_CHUNKS // NS + (s < (N_CHUNKS % NS)).astype(jnp.int32)
    lax.fori_loop(0, n_mine, chunk_body, 0)
    plsc.subcore_barrier()

    pltpu.sync_copy(acc_sh.at[pl.ds(s * STRIPE, STRIPE)],
                    out_hbm.at[pl.ds(base_off + s * STRIPE, STRIPE)])


@jax.jit
def _layer(row, col, w, h):
    return pl.kernel(
        _layer_body,
        out_type=jax.ShapeDtypeStruct((NC * N_PAD, HALF), jnp.float32),
        mesh=plsc.VectorSubcoreMesh(core_axis_name="c", subcore_axis_name="s"),
        scratch_types=[
            pltpu.VMEM((CHUNK,), jnp.int32),
            pltpu.VMEM((CHUNK,), jnp.int32),
            pltpu.VMEM((CHUNK,), jnp.float32),
            pltpu.VMEM((CHUNK, HALF), jnp.float32),
            pltpu.VMEM((ZROWS, HALF), jnp.float32),
            pltpu.VMEM_SHARED((N_PAD, HALF), jnp.float32),
            pltpu.SemaphoreType.DMA,
        ],
        compiler_params=pltpu.CompilerParams(use_tc_tiling_on_sc=False),
    )(row, col, w, h)


def _combine_body(h0, h1, h2, o):
    s = (h0[...] + h1[...] + h2[...]) * (1.0 / 3.0)
    o[:, :HALF] = s[0]
    o[:, HALF:] = s[1]


@jax.jit
def _combine(h0, h1, h2):
    br = 1000
    spec = pl.BlockSpec((2, br, HALF), lambda i: (0, i, 0))
    return pl.pallas_call(
        _combine_body,
        out_shape=jax.ShapeDtypeStruct((N_NODES, DIM), jnp.float32),
        grid=(N_NODES // br,),
        in_specs=[spec, spec, spec],
        out_specs=pl.BlockSpec((br, DIM), lambda i: (i, 0)),
    )(h0.reshape(NC, N_PAD, HALF), h1.reshape(NC, N_PAD, HALF),
      h2.reshape(NC, N_PAD, HALF))


def kernel(edge_index, edge_weight, user_emb, item_emb):
    row = edge_index[0].astype(jnp.int32)
    col = edge_index[1].astype(jnp.int32)
    emb = jnp.concatenate([user_emb, item_emb], axis=0)
    halves = emb.reshape(N_NODES, NC, HALF).transpose(1, 0, 2)
    h0 = (jnp.zeros((NC, N_PAD, HALF), jnp.float32)
          .at[:, :N_NODES, :].set(halves).reshape(NC * N_PAD, HALF))
    h1 = _layer(row, col, edge_weight, h0)
    h2 = _layer(row, col, edge_weight, h1)
    return _combine(h0, h1, h2)


# speedup vs baseline: 13.1276x; 3.8230x over previous
import jax
import jax.numpy as jnp
from jax import lax
from jax.experimental import pallas as pl
from jax.experimental.pallas import tpu as pltpu
from jax.experimental.pallas import tpu_sc as plsc

N_NODES = 50000
N_PAD = 50048
N_EDGES = 800000
DIM = 64
HALF = 32
NC = 2
NS = 16
CHUNK = 128
KCH = 2
BLK = KCH * CHUNK
NBT = N_EDGES // BLK
E_PAD = NBT * BLK
BASE = NBT // NS
REM = NBT % NS
MAXI = BASE + (1 if REM else 0)
NSE = 4
NSR = 3
UNROLL = 12
STRIPE = N_PAD // NS


def _layer_body(eidx_hbm, ew_hbm, h_hbm, out_hbm, *refs):
    ebufs = refs[0:NSE]
    wbufs = refs[NSE:2 * NSE]
    rbufs = tuple(tuple(refs[2 * NSE + u * KCH + k] for k in range(KCH))
                  for u in range(NSR))
    acc_sh, semA, semG, semS = refs[2 * NSE + NSR * KCH:]
    c = lax.axis_index("c")
    s = lax.axis_index("s")
    n_mine = BASE + (s < REM).astype(jnp.int32)

    def blk_id(j):
        return s + j * NS

    def linear_pair(j, u):
        b = blk_id(j)
        return (pltpu.make_async_copy(eidx_hbm.at[b], ebufs[u], semA.at[u]),
                pltpu.make_async_copy(ew_hbm.at[b], wbufs[u], semA.at[u]))

    def gather_desc(ue, ur, k):
        return pltpu.make_async_copy(h_hbm.at[ebufs[ue].at[c, k]], rbufs[ur][k],
                                     semG.at[ur])

    def scatter_desc(ue, ur, k):
        return pltpu.make_async_copy(rbufs[ur][k], acc_sh.at[ebufs[ue].at[2, k]],
                                     semS.at[ur])

    def A(j, u):
        for d in linear_pair(j, u):
            d.start()

    def M(j, ue, ur):
        for d in linear_pair(j, ue):
            d.wait()
        for k in range(KCH):
            gather_desc(ue, ur, k).start()

    def F(ue, ur):
        for k in range(KCH):
            gather_desc(ue, ur, k).wait()
        wbuf = wbufs[ue]

        def multq(q, _):
            rr = q * 16
            for k in range(KCH):
                rows = rbufs[ur][k]
                wv = wbuf[k, pl.ds(rr, 16)]
                for jj in range(16):
                    w = wv[jj]
                    rows[rr + jj, pl.ds(0, 16)] = rows[rr + jj, pl.ds(0, 16)] * w
                    rows[rr + jj, pl.ds(16, 16)] = rows[rr + jj, pl.ds(16, 16)] * w
            return 0
        lax.fori_loop(0, 8, multq, 0)
        for k in range(KCH):
            scatter_desc(ue, ur, k).start(add=True)

    def D(ue, ur):
        for k in range(KCH):
            scatter_desc(ue, ur, k).wait()

    for j in range(3):
        A(jnp.int32(j), j)

    zeros16 = jnp.zeros((16,), jnp.float32)

    zref = rbufs[0][0]

    def zfill(jj, _):
        zref[jj, pl.ds(0, 16)] = zeros16
        zref[jj, pl.ds(16, 16)] = zeros16
        return 0
    lax.fori_loop(0, CHUNK, zfill, 0)

    zbase = s * STRIPE

    def zacc(m, _):
        pltpu.sync_copy(zref, acc_sh.at[pl.ds(zbase + m * CHUNK, CHUNK)])
        return 0
    lax.fori_loop(0, STRIPE // CHUNK, zacc, 0)
    pltpu.sync_copy(zref.at[pl.ds(0, STRIPE % CHUNK)],
                    acc_sh.at[pl.ds(zbase + (STRIPE // CHUNK) * CHUNK, STRIPE % CHUNK)])
    plsc.subcore_barrier()

    M(jnp.int32(0), 0, 0)
    M(jnp.int32(1), 1, 1)

    NSTEP = -(-(MAXI + 1) // UNROLL)

    def step(t, _):
        for u in range(UNROLL):
            i = t * UNROLL + u

            @pl.when(i < n_mine)
            def _(u=u):
                F(u % NSE, u % NSR)

            @pl.when(jnp.logical_and(i >= 1, i - 1 < n_mine))
            def _(u=u):
                D((u - 1) % NSE, (u - 1) % NSR)

            @pl.when(i + 2 < n_mine)
            def _(i=i, u=u):
                M(i + 2, (u + 2) % NSE, (u + 2) % NSR)

            @pl.when(i + 3 < n_mine)
            def _(i=i, u=u):
                A(i + 3, (u + 3) % NSE)
        return 0
    lax.fori_loop(0, NSTEP, step, 0)

    plsc.subcore_barrier()
    pltpu.sync_copy(acc_sh.at[pl.ds(s * STRIPE, STRIPE)],
                    out_hbm.at[pl.ds(c * N_PAD + s * STRIPE, STRIPE)])


@jax.jit
def _layer(eidx, ew, h):
    return pl.kernel(
        _layer_body,
        out_type=jax.ShapeDtypeStruct((NC * N_PAD, HALF), jnp.float32),
        mesh=plsc.VectorSubcoreMesh(core_axis_name="c", subcore_axis_name="s"),
        scratch_types=(
            [pltpu.VMEM((3, KCH, CHUNK), jnp.int32) for _ in range(NSE)]
            + [pltpu.VMEM((KCH, CHUNK), jnp.float32) for _ in range(NSE)]
            + [pltpu.VMEM((CHUNK, HALF), jnp.float32) for _ in range(NSR * KCH)]
            + [
                pltpu.VMEM_SHARED((N_PAD, HALF), jnp.float32),
                pltpu.SemaphoreType.DMA((NSE,)),
                pltpu.SemaphoreType.DMA((NSR,)),
                pltpu.SemaphoreType.DMA((NSR,)),
            ]
        ),
        compiler_params=pltpu.CompilerParams(use_tc_tiling_on_sc=False),
    )(eidx, ew, h)


def _combine_body(h0, h1, h2, o):
    v = (h0[...] + h1[...] + h2[...]) * (1.0 / 3.0)
    o[:, :HALF] = v[0]
    o[:, HALF:] = v[1]


@jax.jit
def _combine(h0, h1, h2):
    br = 1000
    spec = pl.BlockSpec((2, br, HALF), lambda i: (0, i, 0))
    return pl.pallas_call(
        _combine_body,
        out_shape=jax.ShapeDtypeStruct((N_NODES, DIM), jnp.float32),
        grid=(N_NODES // br,),
        in_specs=[spec, spec, spec],
        out_specs=pl.BlockSpec((br, DIM), lambda i: (i, 0)),
    )(h0.reshape(NC, N_PAD, HALF), h1.reshape(NC, N_PAD, HALF),
      h2.reshape(NC, N_PAD, HALF))


def kernel(edge_index, edge_weight, user_emb, item_emb):
    dst = edge_index[0].astype(jnp.int32)
    col = edge_index[1].astype(jnp.int32)
    eidx = (jnp.stack([col, col + N_PAD, dst], axis=0)
            .reshape(3, NBT, KCH, CHUNK).transpose(1, 0, 2, 3))
    ew = edge_weight.reshape(NBT, KCH, CHUNK)

    emb = jnp.concatenate([user_emb, item_emb], axis=0)
    halves = emb.reshape(N_NODES, NC, HALF).transpose(1, 0, 2)
    h0 = (jnp.zeros((NC, N_PAD, HALF), jnp.float32)
          .at[:, :N_NODES, :].set(halves).reshape(NC * N_PAD, HALF))
    h1 = _layer(eidx, ew, h0)
    h2 = _layer(eidx, ew, h1)
    return _combine(h0, h1, h2)
